# baseline (speedup 1.0000x reference)
import functools

import jax
import jax.numpy as jnp
from jax import lax
from jax.experimental import pallas as pl
from jax.experimental.pallas import tpu as pltpu
from jax.experimental.pallas import tpu_sc as plsc

EMB = 96
EMB_PAD = 128
PD = 32
NM, ND, NH = 13, 32, 24
NROWS = NM * ND * NH
DCH = 8
RPG = DCH * NH
TWO_PI = 6.283185307179586

SC_NC = 2
SC_NS = 16
LANES = 16
NW = SC_NC * SC_NS
GCH = 128
NREP = NW * NH
TROWS = NROWS + NREP


def _cdot(a, b):
    return lax.dot_general(a, b, (((1,), (1,)), ((), ())),
                           preferred_element_type=jnp.float32)


def _feats(n, period):
    v = lax.broadcasted_iota(jnp.int32, (n, PD // 2), 0).astype(jnp.float32) / period
    k = lax.broadcasted_iota(jnp.int32, (n, PD // 2), 1).astype(jnp.float32) + 1.0
    ang = TWO_PI * v * k
    return jnp.concatenate([jnp.sin(ang), jnp.cos(ang)], axis=-1)


def _table_body(mw_ref, dw_ref, hw_ref, w_ref, b_ref, out_ref, mt, dt, ht):
    g = pl.program_id(0)

    @pl.when(g == 0)
    def _init():
        w = w_ref[...]
        mt[...] = mw_ref[...] + _cdot(_feats(NM, 12.0), w[:, 0:PD])
        dt[...] = dw_ref[...] + _cdot(_feats(ND, 31.0), w[:, PD:2 * PD])
        ht[...] = (hw_ref[...] + _cdot(_feats(NH, 24.0), w[:, 2 * PD:3 * PD])
                   + b_ref[...])

    @pl.when(g < NM)
    def _combo():
        mrow = mt[pl.ds(g, 1), :]
        hall = ht[...]
        for dc in range(ND // DCH):
            dchunk = dt[pl.ds(dc * DCH, DCH), :]
            combo = dchunk[:, None, :] + hall[None, :, :] + mrow[None, :, :]
            out_ref[pl.ds(dc * RPG, RPG), :] = combo.reshape(RPG, EMB)

    @pl.when(g >= NM)
    def _replicas():
        head = mt[pl.ds(0, 1), :] + dt[pl.ds(0, 1), :] + ht[...]
        rep = jnp.broadcast_to(head[None, :, :], (RPG // NH, NH, EMB))
        rep = rep.reshape(RPG, EMB)
        for dc in range(ND // DCH):
            out_ref[pl.ds(dc * RPG, RPG), :] = rep


def _build_table(month_w, day_w, hour_w, proj_w, proj_b2d):
    grid = (TROWS // (4 * RPG),)
    return pl.pallas_call(
        _table_body,
        grid=grid,
        in_specs=[
            pl.BlockSpec((NM, EMB), lambda g: (0, 0)),
            pl.BlockSpec((ND, EMB), lambda g: (0, 0)),
            pl.BlockSpec((NH, EMB), lambda g: (0, 0)),
            pl.BlockSpec((EMB, EMB), lambda g: (0, 0)),
            pl.BlockSpec((1, EMB), lambda g: (0, 0)),
        ],
        out_specs=pl.BlockSpec((4 * RPG, EMB), lambda g: (g, 0)),
        out_shape=jax.ShapeDtypeStruct((TROWS, EMB), jnp.float32),
        scratch_shapes=[
            pltpu.VMEM((NM, EMB), jnp.float32),
            pltpu.VMEM((ND, EMB), jnp.float32),
            pltpu.VMEM((NH, EMB), jnp.float32),
        ],
        compiler_params=pltpu.CompilerParams(
            dimension_semantics=("arbitrary",)),
    )(month_w, day_w, hour_w, proj_w, proj_b2d)


@functools.lru_cache(maxsize=None)
def _make_sc_gather(batch):
    bpw = batch // NW
    nch = bpw // GCH
    mesh = plsc.VectorSubcoreMesh(core_axis_name="c", subcore_axis_name="s")

    @functools.partial(
        pl.kernel,
        out_type=jax.ShapeDtypeStruct((batch, EMB), jnp.float32),
        mesh=mesh,
        compiler_params=pltpu.CompilerParams(use_tc_tiling_on_sc=False),
        scratch_types=[
            pltpu.VMEM((bpw,), jnp.int32),
            pltpu.VMEM((nch, GCH), jnp.int32),
            pltpu.VMEM((bpw, EMB), jnp.float32),
            pltpu.SemaphoreType.DMA,
        ],
    )
    def sc_gather(x_ref, table_ref, out_ref, xv, idx, rows, sem):
        wid = lax.axis_index("s") * SC_NC + lax.axis_index("c")
        base = wid * bpw
        pltpu.sync_copy(x_ref.at[pl.ds(base, bpw)], xv)
        c100 = jnp.full((LANES,), 100, jnp.int32)
        chmax = jnp.full((LANES,), NH - 1, jnp.int32)
        cdmax = jnp.full((LANES,), ND - 1, jnp.int32)
        cmmax = jnp.full((LANES,), NM - 1, jnp.int32)
        cnh = jnp.full((LANES,), NH, jnp.int32)
        cndh = jnp.full((LANES,), ND * NH, jnp.int32)
        chalf = jnp.full((LANES,), 0.5, jnp.float32)
        crecip = jnp.full((LANES,), 0.01, jnp.float32)
        cpr = GCH // LANES

        def div100(vf):
            return ((vf + chalf) * crecip).astype(jnp.int32)

        repoff = jnp.full((LANES,), NROWS, jnp.int32) + lax.broadcast(
            wid * NH, (LANES,))

        def dec(j, carry):
            v = xv[pl.ds(j * LANES, LANES)]
            vf = v.astype(jnp.float32)
            q1 = div100(vf)
            q2 = div100(q1.astype(jnp.float32))
            h = lax.min(v - q1 * c100, chmax)
            d = lax.min(q1 - q2 * c100, cdmax)
            m = lax.min(q2, cmmax)
            cidx = m * cndh + d * cnh + h
            cidx = lax.select(cidx < cnh, repoff + cidx, cidx)
            idx[j // cpr, pl.ds((j % cpr) * LANES, LANES)] = cidx
            return carry

        lax.fori_loop(0, bpw // LANES, dec, 0)
        copies = [
            pltpu.async_copy(table_ref.at[idx.at[c]],
                             rows.at[pl.ds(c * GCH, GCH)], sem)
            for c in range(nch)
        ]
        for cp in copies:
            cp.wait()
        pltpu.sync_copy(rows, out_ref.at[pl.ds(base, bpw)])

    return sc_gather


def kernel(x, year_w, month_w, day_w, hour_w, proj_w, proj_b):
    del year_w
    x0 = x.reshape(-1).astype(jnp.int32)
    table = _build_table(month_w, day_w, hour_w, proj_w,
                         proj_b.reshape(1, EMB))
    return _make_sc_gather(x0.shape[0])(x0, table)


# speedup vs baseline: 1.3498x; 1.3498x over previous
import functools

import jax
import jax.numpy as jnp
from jax import lax
from jax.experimental import pallas as pl
from jax.experimental.pallas import tpu as pltpu
from jax.experimental.pallas import tpu_sc as plsc

EMB = 96
EMB_PAD = 128
PD = 32
NM, ND, NH = 13, 32, 24
NROWS = NM * ND * NH
DCH = 8
RPG = DCH * NH
TWO_PI = 6.283185307179586

SC_NC = 2
SC_NS = 16
LANES = 16
NW = SC_NC * SC_NS
GCH = 128
NREP = NW * NH
TROWS = NROWS + NREP


def _cdot(a, b):
    return lax.dot_general(a, b, (((1,), (1,)), ((), ())),
                           preferred_element_type=jnp.float32)


def _feats(n, period):
    v = lax.broadcasted_iota(jnp.int32, (n, PD // 2), 0).astype(jnp.float32) / period
    k = lax.broadcasted_iota(jnp.int32, (n, PD // 2), 1).astype(jnp.float32) + 1.0
    ang = TWO_PI * v * k
    return jnp.concatenate([jnp.sin(ang), jnp.cos(ang)], axis=-1)


def _table_body(mw_ref, dw_ref, hw_ref, w_ref, b_ref, out_ref, mt, dt, ht):
    g = pl.program_id(0)

    @pl.when(g == 0)
    def _init():
        w = w_ref[...]
        mt[...] = mw_ref[...] + _cdot(_feats(NM, 12.0), w[:, 0:PD])
        dt[...] = dw_ref[...] + _cdot(_feats(ND, 31.0), w[:, PD:2 * PD])
        ht[...] = (hw_ref[...] + _cdot(_feats(NH, 24.0), w[:, 2 * PD:3 * PD])
                   + b_ref[...])

    zpad = jnp.zeros((RPG, EMB_PAD - EMB), jnp.float32)

    hall = ht[...]
    for half in range(2):
        m = g * 2 + half

        @pl.when(m < NM)
        def _combo(m=m, half=half):
            mrow = mt[pl.ds(m, 1), :]
            for dc in range(ND // DCH):
                dchunk = dt[pl.ds(dc * DCH, DCH), :]
                combo = (dchunk[:, None, :] + hall[None, :, :]
                         + mrow[None, :, :])
                out_ref[pl.ds((half * 4 + dc) * RPG, RPG), :] = (
                    jnp.concatenate([combo.reshape(RPG, EMB), zpad],
                                    axis=-1))

        @pl.when(m >= NM)
        def _replicas(half=half):
            head = (mt[pl.ds(0, 1), :] + dt[pl.ds(0, 1), :] + hall)
            rep = jnp.broadcast_to(head[None, :, :], (RPG // NH, NH, EMB))
            rep = jnp.concatenate([rep.reshape(RPG, EMB), zpad], axis=-1)
            for dc in range(ND // DCH):
                out_ref[pl.ds((half * 4 + dc) * RPG, RPG), :] = rep


def _build_table(month_w, day_w, hour_w, proj_w, proj_b2d):
    grid = (TROWS // (8 * RPG),)
    return pl.pallas_call(
        _table_body,
        grid=grid,
        in_specs=[
            pl.BlockSpec((NM, EMB), lambda g: (0, 0)),
            pl.BlockSpec((ND, EMB), lambda g: (0, 0)),
            pl.BlockSpec((NH, EMB), lambda g: (0, 0)),
            pl.BlockSpec((EMB, EMB), lambda g: (0, 0)),
            pl.BlockSpec((1, EMB), lambda g: (0, 0)),
        ],
        out_specs=pl.BlockSpec((8 * RPG, EMB_PAD), lambda g: (g, 0)),
        out_shape=jax.ShapeDtypeStruct((TROWS, EMB_PAD), jnp.float32),
        scratch_shapes=[
            pltpu.VMEM((NM, EMB), jnp.float32),
            pltpu.VMEM((ND, EMB), jnp.float32),
            pltpu.VMEM((NH, EMB), jnp.float32),
        ],
        compiler_params=pltpu.CompilerParams(
            dimension_semantics=("arbitrary",)),
    )(month_w, day_w, hour_w, proj_w, proj_b2d)


@functools.lru_cache(maxsize=None)
def _make_sc_gather(batch):
    bpw = batch // NW
    nch = bpw // GCH
    mesh = plsc.VectorSubcoreMesh(core_axis_name="c", subcore_axis_name="s")

    @functools.partial(
        pl.kernel,
        out_type=jax.ShapeDtypeStruct((batch, EMB_PAD), jnp.float32),
        mesh=mesh,
        scratch_types=[
            pltpu.VMEM((bpw,), jnp.int32),
            pltpu.VMEM((nch, GCH), jnp.int32),
            pltpu.VMEM((bpw, EMB_PAD), jnp.float32),
            pltpu.SemaphoreType.DMA,
            pltpu.SemaphoreType.DMA,
        ],
    )
    def sc_gather(x_ref, table_ref, out_ref, xv, idx, rows, sem, wsem):
        wid = lax.axis_index("s") * SC_NC + lax.axis_index("c")
        base = wid * bpw
        pltpu.sync_copy(x_ref.at[pl.ds(base, bpw)], xv)
        c100 = jnp.full((LANES,), 100, jnp.int32)
        chmax = jnp.full((LANES,), NH - 1, jnp.int32)
        cdmax = jnp.full((LANES,), ND - 1, jnp.int32)
        cmmax = jnp.full((LANES,), NM - 1, jnp.int32)
        cnh = jnp.full((LANES,), NH, jnp.int32)
        cndh = jnp.full((LANES,), ND * NH, jnp.int32)
        chalf = jnp.full((LANES,), 0.5, jnp.float32)
        crecip = jnp.full((LANES,), 0.01, jnp.float32)
        cpr = GCH // LANES

        def div100(vf):
            return ((vf + chalf) * crecip).astype(jnp.int32)

        repoff = jnp.full((LANES,), NROWS, jnp.int32) + lax.broadcast(
            wid * NH, (LANES,))

        def dec(j, carry):
            v = xv[pl.ds(j * LANES, LANES)]
            vf = v.astype(jnp.float32)
            q1 = div100(vf)
            q2 = div100(q1.astype(jnp.float32))
            h = lax.min(v - q1 * c100, chmax)
            d = lax.min(q1 - q2 * c100, cdmax)
            m = lax.min(q2, cmmax)
            cidx = m * cndh + d * cnh + h
            cidx = lax.select(cidx < cnh, repoff + cidx, cidx)
            idx[j // cpr, pl.ds((j % cpr) * LANES, LANES)] = cidx
            return carry

        lax.fori_loop(0, bpw // LANES, dec, 0)
        gathers = [
            pltpu.async_copy(table_ref.at[idx.at[c]],
                             rows.at[pl.ds(c * GCH, GCH)], sem)
            for c in range(nch)
        ]
        writes = []
        for c in range(nch):
            gathers[c].wait()
            writes.append(
                pltpu.async_copy(rows.at[pl.ds(c * GCH, GCH)],
                                 out_ref.at[pl.ds(base + c * GCH, GCH)],
                                 wsem))
        for wr in writes:
            wr.wait()

    return sc_gather


def kernel(x, year_w, month_w, day_w, hour_w, proj_w, proj_b):
    del year_w
    x0 = x.reshape(-1).astype(jnp.int32)
    table = _build_table(month_w, day_w, hour_w, proj_w,
                         proj_b.reshape(1, EMB))
    padded = _make_sc_gather(x0.shape[0])(x0, table)
    return padded[:, :EMB]
